# baseline (speedup 1.0000x reference)
import functools

import jax
import jax.numpy as jnp
from jax import lax
from jax.experimental import pallas as pl
from jax.experimental.pallas import tpu as pltpu
from jax.experimental.pallas import tpu_sc as plsc

_B, _T, _D = 4, 2048, 1024
_NC, _NS = 2, 16
_NW = _NC * _NS
_TPW = _T // _NW
_R = 16
_NSUB = _TPW // _R
_UNROLL = 8
_CPR = _D // 16


def _sc_body(feat_hbm, sin_hbm, out_hbm, sin_v, feat_v):
    wid = lax.axis_index("s") * _NC + lax.axis_index("c")
    t0 = wid * _TPW

    def add_row(r, carry):
        def add_block(kk, carry2):
            for j in range(_UNROLL):
                sl = pl.ds((kk * _UNROLL + j) * 16, 16)
                plsc.addupdate(feat_v.at[r, sl], sin_v[r, sl])
            return carry2
        return lax.fori_loop(0, _CPR // _UNROLL, add_block, carry)

    for s in range(_NSUB):
        row = t0 + s * _R
        pltpu.sync_copy(sin_hbm.at[pl.ds(row, _R)], sin_v)
        for b in range(_B):
            pltpu.sync_copy(feat_hbm.at[b, pl.ds(row, _R)], feat_v)
            lax.fori_loop(0, _R, add_row, 0)
            pltpu.sync_copy(feat_v, out_hbm.at[b, pl.ds(row, _R)])


_sc_kernel = functools.partial(
    pl.kernel,
    out_type=jax.ShapeDtypeStruct((_B, _T, _D), jnp.float32),
    mesh=plsc.VectorSubcoreMesh(core_axis_name="c", subcore_axis_name="s"),
    scratch_types=[
        pltpu.VMEM((_R, _D), jnp.float32),
        pltpu.VMEM((_R, _D), jnp.float32),
    ],
)(_sc_body)


def kernel(features, sinusoids):
    return _sc_kernel(features, sinusoids)
